# baseline (speedup 1.0000x reference)
import functools
import jax
import jax.numpy as jnp
from jax import lax
from jax.experimental import pallas as pl
from jax.experimental.pallas import tpu as pltpu
from jax.experimental.pallas import tpu_sc as plsc

_NC = 2
_NS = 16
_NW = _NC * _NS
_L = 16
_D = 64
_C = 128
_NBUF = 4
_SCALE = 8.0


def _make_sc_lookup(nsl):
    assert nsl % _NBUF == 0
    mesh = plsc.VectorSubcoreMesh(
        core_axis_name="c", subcore_axis_name="s",
        num_cores=_NC, num_subcores=_NS)

    @functools.partial(
        pl.kernel,
        mesh=mesh,
        out_type=jax.ShapeDtypeStruct((200, 8, 32, 8, _C), jnp.float32),
        scratch_types=[
            pltpu.VMEM((nsl, _C), jnp.int32),
            pltpu.VMEM((_NBUF, _C, _D), jnp.float32),
            pltpu.VMEM((_NBUF, 8, 8, _C), jnp.float32),
            pltpu.SemaphoreType.DMA,
            pltpu.SemaphoreType.DMA,
        ],
        compiler_params=pltpu.CompilerParams(
            use_tc_tiling_on_sc=False, needs_layout_passes=False),
    )
    def sc_lookup(x_hbm, table_hbm, out_hbm, idx_v, gbuf, sbuf, gsem, ssem):
        wid = lax.axis_index("s") * _NC + lax.axis_index("c")
        pltpu.sync_copy(x_hbm.at[wid], idx_v)
        iota = lax.iota(jnp.int32, _L)
        ivs = [iota + (cb * _L) for cb in range(_C // _L)]

        def slab_coords(m):
            g = wid * nsl + m
            j = (g >> 8) * 8 + (g & 7)
            ti = (g >> 3) & 31
            return j, ti

        for b in range(_NBUF):
            pltpu.async_copy(table_hbm.at[idx_v.at[b]], gbuf.at[b], gsem)

        def group(grp, carry):
            for b in range(_NBUF):
                m = grp * _NBUF + b
                pltpu.make_async_copy(
                    table_hbm.at[idx_v.at[0]], gbuf.at[b], gsem).wait()
                @pl.when(grp > 0)
                def _():
                    pltpu.make_async_copy(
                        sbuf.at[b], out_hbm.at[0, :, 0], ssem).wait()

                @plsc.parallel_loop(0, _D, 1, unroll=8)
                def _(d):
                    dv = jnp.full((_L,), d, jnp.int32)
                    for cb in range(_C // _L):
                        v = plsc.load_gather(gbuf.at[b], [ivs[cb], dv])
                        sbuf[b, d >> 3, d & 7,
                             pl.ds(cb * _L, _L)] = v * _SCALE

                j, ti = slab_coords(m)
                pltpu.async_copy(sbuf.at[b], out_hbm.at[j, :, ti], ssem)
                @pl.when(m + _NBUF < nsl)
                def _():
                    pltpu.async_copy(
                        table_hbm.at[idx_v.at[m + _NBUF]], gbuf.at[b], gsem)
            return carry

        lax.fori_loop(0, nsl // _NBUF, group, 0)
        for b in range(_NBUF):
            pltpu.make_async_copy(
                sbuf.at[b], out_hbm.at[0, :, 0], ssem).wait()

    return sc_lookup


def kernel(x, table):
    bsz, seq = x.shape
    assert (bsz, seq) == (4096, 200)
    xp = (x.astype(jnp.int32).T
          .reshape(25, 8, 32, _C)
          .transpose(0, 2, 1, 3)
          .reshape(_NW, 200, _C))
    out_p = _make_sc_lookup(200)(xp, table)
    out = out_p.transpose(2, 4, 0, 1, 3).reshape(bsz, seq, _D)
    return out


# speedup vs baseline: 1.7129x; 1.7129x over previous
import functools
import jax
import jax.numpy as jnp
from jax import lax
from jax.experimental import pallas as pl
from jax.experimental.pallas import tpu as pltpu
from jax.experimental.pallas import tpu_sc as plsc

_NC = 2
_NS = 16
_NW = _NC * _NS
_L = 16
_D = 64
_C = 128
_NBUF = 4
_SCALE = 8.0


def _make_sc_lookup(nsl):
    assert nsl % _NBUF == 0
    mesh = plsc.VectorSubcoreMesh(
        core_axis_name="c", subcore_axis_name="s",
        num_cores=_NC, num_subcores=_NS)

    @functools.partial(
        pl.kernel,
        mesh=mesh,
        out_type=jax.ShapeDtypeStruct((200, 8, 32, 8, _C), jnp.float32),
        scratch_types=[
            pltpu.VMEM((nsl, _C), jnp.int32),
            pltpu.VMEM((_NBUF, _C, _D), jnp.float32),
            pltpu.VMEM((_NBUF, 8, 8, _C + 1), jnp.float32),
            pltpu.SemaphoreType.DMA,
            pltpu.SemaphoreType.DMA,
        ],
        compiler_params=pltpu.CompilerParams(
            use_tc_tiling_on_sc=False, needs_layout_passes=False),
    )
    def sc_lookup(x_hbm, table_hbm, out_hbm, idx_v, gbuf, sbuf, gsem, ssem):
        wid = lax.axis_index("s") * _NC + lax.axis_index("c")
        pltpu.sync_copy(x_hbm.at[wid], idx_v)
        iota = lax.iota(jnp.int32, _L)
        dvs = [iota + (cb * _L) for cb in range(_D // _L)]
        tdvs = [dv >> 3 for dv in dvs]
        rvs = [dv & 7 for dv in dvs]

        def slab_coords(m):
            g = wid * nsl + m
            j = (g >> 8) * 8 + (g & 7)
            ti = (g >> 3) & 31
            return j, ti

        for b in range(_NBUF):
            pltpu.async_copy(table_hbm.at[idx_v.at[b]], gbuf.at[b], gsem)

        def group(grp, carry):
            for b in range(_NBUF):
                m = grp * _NBUF + b
                pltpu.make_async_copy(
                    table_hbm.at[idx_v.at[0]], gbuf.at[b], gsem).wait()
                @pl.when(grp > 0)
                def _():
                    pltpu.make_async_copy(
                        sbuf.at[b, :, :, pl.ds(0, _C)],
                        out_hbm.at[0, :, 0], ssem).wait()

                @plsc.parallel_loop(0, _C, 1, unroll=4)
                def _(i):
                    iv = jnp.full((_L,), i, jnp.int32)
                    for cb in range(_D // _L):
                        v = gbuf[b, i, pl.ds(cb * _L, _L)] * _SCALE
                        plsc.store_scatter(
                            sbuf.at[b], [tdvs[cb], rvs[cb], iv], v)

                j, ti = slab_coords(m)
                pltpu.async_copy(
                    sbuf.at[b, :, :, pl.ds(0, _C)],
                    out_hbm.at[j, :, ti], ssem)
                @pl.when(m + _NBUF < nsl)
                def _():
                    pltpu.async_copy(
                        table_hbm.at[idx_v.at[m + _NBUF]], gbuf.at[b], gsem)
            return carry

        lax.fori_loop(0, nsl // _NBUF, group, 0)
        for b in range(_NBUF):
            pltpu.make_async_copy(
                sbuf.at[b, :, :, pl.ds(0, _C)],
                out_hbm.at[0, :, 0], ssem).wait()

    return sc_lookup


def kernel(x, table):
    bsz, seq = x.shape
    assert (bsz, seq) == (4096, 200)
    xp = (x.astype(jnp.int32).T
          .reshape(25, 8, 32, _C)
          .transpose(0, 2, 1, 3)
          .reshape(_NW, 200, _C))
    out_p = _make_sc_lookup(200)(xp, table)
    out = out_p.transpose(2, 4, 0, 1, 3).reshape(bsz, seq, _D)
    return out
